# baseline (speedup 1.0000x reference)
import jax
import jax.numpy as jnp
from jax.experimental import pallas as pl
from jax.experimental.pallas import tpu as pltpu

_NUM_CLASSES = 5


def _rank_loss_kernel(scol_ref, lcol_ref, srow_ref, lrow_ref,
                      lfull_ref, out_ref, acc_ref):
    b = pl.program_id(0)
    rt = pl.program_id(1)
    ct = pl.program_id(2)
    nb = pl.num_programs(0)
    nt = pl.num_programs(1)
    tr = scol_ref.shape[0]
    g = lfull_ref.shape[2]

    @pl.when(jnp.logical_and(b == 0, jnp.logical_and(rt == 0, ct == 0)))
    def _init_totals():
        acc_ref[2] = 0.0
        acc_ref[3] = 0.0

    @pl.when(jnp.logical_and(rt == 0, ct == 0))
    def _init_group():
        acc_ref[0] = 0.0
        lab = lfull_ref[0]
        sumsq = jnp.zeros((), jnp.float32)
        for a in range(_NUM_CLASSES):
            cnt = jnp.sum(jnp.where(lab == a, 1.0, 0.0))
            sumsq += cnt * cnt
        acc_ref[1] = (float(g * g) - sumsq) * 0.5

    @pl.when(rt >= ct)
    def _tile():
        scol = scol_ref[...]
        lcol = lcol_ref[...]
        srow = srow_ref[0]
        lrow = lrow_ref[0]
        d = scol - srow
        f = jnp.where(lcol > lrow, -d, d)
        sp = jnp.maximum(f, 0.0) + jnp.log1p(jnp.exp(-jnp.abs(d)))
        gr = rt * tr + jax.lax.broadcasted_iota(jnp.int32, d.shape, 0)
        gc = ct * tr + jax.lax.broadcasted_iota(jnp.int32, d.shape, 1)
        mask = jnp.logical_and(lcol != lrow, gr > gc)
        acc_ref[0] += jnp.sum(jnp.where(mask, sp, 0.0))

    @pl.when(jnp.logical_and(rt == nt - 1, ct == nt - 1))
    def _finalize_group():
        n_pairs = acc_ref[1]
        safe_n = jnp.where(n_pairs > 0, n_pairs, 1.0)
        acc_ref[2] += jnp.where(n_pairs > 0, acc_ref[0] / safe_n, 0.0)
        acc_ref[3] += jnp.where(n_pairs > 0, 1.0, 0.0)

        @pl.when(b == nb - 1)
        def _finalize_output():
            count = acc_ref[3]
            safe_c = jnp.where(count > 0, count, 1.0)
            out_ref[0, 0] = jnp.where(count > 0, acc_ref[2] / safe_c, 0.0)


def kernel(scores, labels, group_sizes):
    scores = scores.reshape(-1)
    labels = labels.reshape(-1)
    n = scores.shape[0]
    num_groups = group_sizes.shape[0]
    g = n // num_groups
    tr = 256
    nt = g // tr

    scores_3d = scores.reshape(num_groups, 1, g)
    labels_3d = labels.reshape(num_groups, 1, g)
    scores_col = scores.reshape(n, 1)
    labels_col = labels.reshape(n, 1)

    out = pl.pallas_call(
        _rank_loss_kernel,
        grid=(num_groups, nt, nt),
        in_specs=[
            pl.BlockSpec((tr, 1), lambda b, rt, ct: (b * nt + rt, 0)),
            pl.BlockSpec((tr, 1), lambda b, rt, ct: (b * nt + rt, 0)),
            pl.BlockSpec((1, 1, tr), lambda b, rt, ct: (b, 0, ct)),
            pl.BlockSpec((1, 1, tr), lambda b, rt, ct: (b, 0, ct)),
            pl.BlockSpec((1, 1, g), lambda b, rt, ct: (b, 0, 0)),
        ],
        out_specs=pl.BlockSpec(memory_space=pltpu.SMEM),
        out_shape=jax.ShapeDtypeStruct((1, 1), jnp.float32),
        scratch_shapes=[pltpu.SMEM((4,), jnp.float32)],
    )(scores_col, labels_col, scores_3d, labels_3d, labels_3d)
    return out[0, 0]


# speedup vs baseline: 2.7813x; 2.7813x over previous
import jax
import jax.numpy as jnp
from jax.experimental import pallas as pl
from jax.experimental.pallas import tpu as pltpu

_NUM_CLASSES = 5


def _rank_loss_kernel(sca_ref, scb_ref, lca_ref, lcb_ref,
                      srow_ref, srev_ref, lrow_ref, lrev_ref,
                      out_ref, acc_ref):
    b = pl.program_id(0)
    rt = pl.program_id(1)
    nb = pl.num_programs(0)
    nt = pl.num_programs(1)
    tr = sca_ref.shape[0]
    g = lrow_ref.shape[2]

    @pl.when(jnp.logical_and(b == 0, rt == 0))
    def _init_totals():
        acc_ref[2] = 0.0
        acc_ref[3] = 0.0

    @pl.when(rt == 0)
    def _init_group():
        acc_ref[0] = 0.0
        lab = lrow_ref[0]
        sumsq = jnp.zeros((), jnp.float32)
        for a in range(_NUM_CLASSES):
            cnt = jnp.sum(jnp.where(lab == a, 1.0, 0.0))
            sumsq += cnt * cnt
        acc_ref[1] = (float(g * g) - sumsq) * 0.5

    sca = sca_ref[...]
    scb = scb_ref[...]
    lca = lca_ref[...]
    lcb = lcb_ref[...]
    srow = srow_ref[0]
    srev = srev_ref[0]
    lrow = lrow_ref[0]
    lrev = lrev_ref[0]

    shape = (tr, g)
    r = rt * tr + jax.lax.broadcasted_iota(jnp.int32, shape, 0)
    j = jax.lax.broadcasted_iota(jnp.int32, shape, 1)
    top = j < r
    d = jnp.where(top, sca - srow, scb - srev)
    lc = jnp.where(top, lca, lcb)
    lr = jnp.where(top, lrow, lrev)
    f = jnp.where(lc > lr, -d, d)
    sp = jnp.maximum(f, 0.0) + jnp.log1p(jnp.exp(-jnp.abs(d)))
    acc_ref[0] += jnp.sum(jnp.where(lc != lr, sp, 0.0))

    @pl.when(rt == nt - 1)
    def _finalize_group():
        n_pairs = acc_ref[1]
        safe_n = jnp.where(n_pairs > 0, n_pairs, 1.0)
        acc_ref[2] += jnp.where(n_pairs > 0, acc_ref[0] / safe_n, 0.0)
        acc_ref[3] += jnp.where(n_pairs > 0, 1.0, 0.0)

        @pl.when(b == nb - 1)
        def _finalize_output():
            count = acc_ref[3]
            safe_c = jnp.where(count > 0, count, 1.0)
            out_ref[0, 0] = jnp.where(count > 0, acc_ref[2] / safe_c, 0.0)


def kernel(scores, labels, group_sizes):
    scores = scores.reshape(-1)
    labels = labels.reshape(-1)
    n = scores.shape[0]
    num_groups = group_sizes.shape[0]
    g = n // num_groups
    h = g // 2
    tr = 256
    nt = h // tr

    s2 = scores.reshape(num_groups, g)
    l2 = labels.reshape(num_groups, g)
    sca = s2[:, :h].reshape(num_groups * h, 1)
    scb = s2[:, :h - 1:-1].reshape(num_groups * h, 1)
    lca = l2[:, :h].reshape(num_groups * h, 1)
    lcb = l2[:, :h - 1:-1].reshape(num_groups * h, 1)
    srow = s2.reshape(num_groups, 1, g)
    srev = s2[:, ::-1].reshape(num_groups, 1, g)
    lrow = l2.reshape(num_groups, 1, g)
    lrev = l2[:, ::-1].reshape(num_groups, 1, g)

    col = pl.BlockSpec((tr, 1), lambda b, rt: (b * nt + rt, 0))
    row = pl.BlockSpec((1, 1, g), lambda b, rt: (b, 0, 0))

    out = pl.pallas_call(
        _rank_loss_kernel,
        grid=(num_groups, nt),
        in_specs=[col, col, col, col, row, row, row, row],
        out_specs=pl.BlockSpec(memory_space=pltpu.SMEM),
        out_shape=jax.ShapeDtypeStruct((1, 1), jnp.float32),
        scratch_shapes=[pltpu.SMEM((4,), jnp.float32)],
    )(sca, scb, lca, lcb, srow, srev, lrow, lrev)
    return out[0, 0]


# speedup vs baseline: 2.8219x; 1.0146x over previous
import jax
import jax.numpy as jnp
from jax.experimental import pallas as pl
from jax.experimental.pallas import tpu as pltpu

_NUM_CLASSES = 5


def _rank_loss_kernel(sca_ref, scb_ref, lca_ref, lcb_ref,
                      srow_ref, srev_ref, lrow_ref, lrev_ref,
                      out_ref, acc_ref):
    b = pl.program_id(0)
    rt = pl.program_id(1)
    nb = pl.num_programs(0)
    nt = pl.num_programs(1)
    tr = sca_ref.shape[0]
    g = lrow_ref.shape[2]

    @pl.when(jnp.logical_and(b == 0, rt == 0))
    def _init_totals():
        acc_ref[3] = 0.0
        acc_ref[4] = 0.0

    @pl.when(rt == 0)
    def _init_group():
        acc_ref[0] = 0.0
        lab = lrow_ref[0]
        s = srow_ref[0]
        sumsq = jnp.zeros((), jnp.float32)
        lin = jnp.zeros((), jnp.float32)
        for a in range(_NUM_CLASSES):
            cnt = jnp.sum(jnp.where(lab == a, 1.0, 0.0))
            sumsq += cnt * cnt
            lin += cnt * jnp.sum(s * jnp.sign(lab - a).astype(jnp.float32))
        acc_ref[1] = (float(g * g) - sumsq) * 0.5
        acc_ref[2] = lin

    sca = sca_ref[...]
    scb = scb_ref[...]
    lca = lca_ref[...]
    lcb = lcb_ref[...]
    srow = srow_ref[0]
    srev = srev_ref[0]
    lrow = lrow_ref[0]
    lrev = lrev_ref[0]

    shape = (tr, g)
    r = rt * tr + jax.lax.broadcasted_iota(jnp.int32, shape, 0)
    j = jax.lax.broadcasted_iota(jnp.int32, shape, 1)
    top = j < r
    d = jnp.where(top, sca - srow, scb - srev)
    a = jnp.abs(d)
    t = jnp.log1p(jnp.exp(-a)) + 0.5 * a
    m = jnp.logical_or(jnp.logical_and(top, lca != lrow),
                       jnp.logical_and(jnp.logical_not(top), lcb != lrev))
    acc_ref[0] += jnp.sum(jnp.where(m, t, 0.0))

    @pl.when(rt == nt - 1)
    def _finalize_group():
        n_pairs = acc_ref[1]
        safe_n = jnp.where(n_pairs > 0, n_pairs, 1.0)
        loss = (acc_ref[0] - 0.5 * acc_ref[2]) / safe_n
        acc_ref[3] += jnp.where(n_pairs > 0, loss, 0.0)
        acc_ref[4] += jnp.where(n_pairs > 0, 1.0, 0.0)

        @pl.when(b == nb - 1)
        def _finalize_output():
            count = acc_ref[4]
            safe_c = jnp.where(count > 0, count, 1.0)
            out_ref[0, 0] = jnp.where(count > 0, acc_ref[3] / safe_c, 0.0)


def kernel(scores, labels, group_sizes):
    scores = scores.reshape(-1)
    labels = labels.reshape(-1)
    n = scores.shape[0]
    num_groups = group_sizes.shape[0]
    g = n // num_groups
    h = g // 2
    tr = 256
    nt = h // tr

    s2 = scores.reshape(num_groups, g)
    l2 = labels.reshape(num_groups, g)
    sca = s2[:, :h].reshape(num_groups * h, 1)
    scb = s2[:, :h - 1:-1].reshape(num_groups * h, 1)
    lca = l2[:, :h].reshape(num_groups * h, 1)
    lcb = l2[:, :h - 1:-1].reshape(num_groups * h, 1)
    srow = s2.reshape(num_groups, 1, g)
    srev = s2[:, ::-1].reshape(num_groups, 1, g)
    lrow = l2.reshape(num_groups, 1, g)
    lrev = l2[:, ::-1].reshape(num_groups, 1, g)

    col = pl.BlockSpec((tr, 1), lambda b, rt: (b * nt + rt, 0))
    row = pl.BlockSpec((1, 1, g), lambda b, rt: (b, 0, 0))

    out = pl.pallas_call(
        _rank_loss_kernel,
        grid=(num_groups, nt),
        in_specs=[col, col, col, col, row, row, row, row],
        out_specs=pl.BlockSpec(memory_space=pltpu.SMEM),
        out_shape=jax.ShapeDtypeStruct((1, 1), jnp.float32),
        scratch_shapes=[pltpu.SMEM((5,), jnp.float32)],
    )(sca, scb, lca, lcb, srow, srev, lrow, lrev)
    return out[0, 0]


# speedup vs baseline: 3.7173x; 1.3173x over previous
import jax
import jax.numpy as jnp
from jax.experimental import pallas as pl
from jax.experimental.pallas import tpu as pltpu

_NUM_CLASSES = 5


def _rank_loss_kernel(sca_ref, scb_ref, lca_ref, lcb_ref,
                      srow_ref, srev_ref, lrow_ref, lrev_ref,
                      out_ref, acc_ref):
    b = pl.program_id(0)
    rt = pl.program_id(1)
    nb = pl.num_programs(0)
    nt = pl.num_programs(1)
    tr = sca_ref.shape[0]
    g = lrow_ref.shape[2]

    @pl.when(jnp.logical_and(b == 0, rt == 0))
    def _init_totals():
        acc_ref[3] = 0.0
        acc_ref[4] = 0.0

    @pl.when(rt == 0)
    def _init_group():
        acc_ref[0] = 0.0
        lab = lrow_ref[0]
        s = srow_ref[0]
        sumsq = jnp.zeros((), jnp.float32)
        lin = jnp.zeros((), jnp.float32)
        for a in range(_NUM_CLASSES):
            cnt = jnp.sum(jnp.where(lab == a, 1.0, 0.0))
            sumsq += cnt * cnt
            lin += cnt * jnp.sum(s * jnp.sign(lab - a).astype(jnp.float32))
        acc_ref[1] = (float(g * g) - sumsq) * 0.5
        acc_ref[2] = lin

    sca = sca_ref[...]
    scb = scb_ref[...]
    lca = lca_ref[...]
    lcb = lcb_ref[...]
    srow = srow_ref[0]
    srev = srev_ref[0]
    lrow = lrow_ref[0]
    lrev = lrev_ref[0]

    shape = (tr, g)
    r = rt * tr + jax.lax.broadcasted_iota(jnp.int32, shape, 0)
    j = jax.lax.broadcasted_iota(jnp.int32, shape, 1)
    top = j < r
    d = jnp.where(top, sca - srow, scb - srev)
    a = jnp.abs(d)
    t = jnp.log1p(jnp.exp(-a)) + 0.5 * a
    m = jnp.logical_or(jnp.logical_and(top, lca != lrow),
                       jnp.logical_and(jnp.logical_not(top), lcb != lrev))
    acc_ref[0] += jnp.sum(jnp.where(m, t, 0.0))

    @pl.when(rt == nt - 1)
    def _finalize_group():
        n_pairs = acc_ref[1]
        safe_n = jnp.where(n_pairs > 0, n_pairs, 1.0)
        loss = (acc_ref[0] - 0.5 * acc_ref[2]) / safe_n
        acc_ref[3] += jnp.where(n_pairs > 0, loss, 0.0)
        acc_ref[4] += jnp.where(n_pairs > 0, 1.0, 0.0)

        @pl.when(b == nb - 1)
        def _finalize_output():
            count = acc_ref[4]
            safe_c = jnp.where(count > 0, count, 1.0)
            out_ref[0, 0] = jnp.where(count > 0, acc_ref[3] / safe_c, 0.0)


def kernel(scores, labels, group_sizes):
    scores = scores.reshape(-1)
    labels = labels.reshape(-1)
    n = scores.shape[0]
    num_groups = group_sizes.shape[0]
    g = n // num_groups
    h = g // 2
    tr = 512
    nt = h // tr

    s2 = scores.reshape(num_groups, g)
    l2 = labels.reshape(num_groups, g)
    sca = s2[:, :h].reshape(num_groups * h, 1)
    scb = s2[:, :h - 1:-1].reshape(num_groups * h, 1)
    lca = l2[:, :h].reshape(num_groups * h, 1)
    lcb = l2[:, :h - 1:-1].reshape(num_groups * h, 1)
    srow = s2.reshape(num_groups, 1, g)
    srev = s2[:, ::-1].reshape(num_groups, 1, g)
    lrow = l2.reshape(num_groups, 1, g)
    lrev = l2[:, ::-1].reshape(num_groups, 1, g)

    col = pl.BlockSpec((tr, 1), lambda b, rt: (b * nt + rt, 0))
    row = pl.BlockSpec((1, 1, g), lambda b, rt: (b, 0, 0))

    out = pl.pallas_call(
        _rank_loss_kernel,
        grid=(num_groups, nt),
        in_specs=[col, col, col, col, row, row, row, row],
        out_specs=pl.BlockSpec(memory_space=pltpu.SMEM),
        out_shape=jax.ShapeDtypeStruct((1, 1), jnp.float32),
        scratch_shapes=[pltpu.SMEM((5,), jnp.float32)],
    )(sca, scb, lca, lcb, srow, srev, lrow, lrev)
    return out[0, 0]


# speedup vs baseline: 4.2443x; 1.1418x over previous
import jax
import jax.numpy as jnp
from jax.experimental import pallas as pl
from jax.experimental.pallas import tpu as pltpu

_NUM_CLASSES = 5


def _rank_loss_kernel(sca_ref, scb_ref, lca_ref, lcb_ref,
                      srow_ref, srev_ref, lrow_ref, lrev_ref,
                      out_ref, acc_ref):
    b = pl.program_id(0)
    rt = pl.program_id(1)
    nb = pl.num_programs(0)
    nt = pl.num_programs(1)
    tr = sca_ref.shape[0]
    g = lrow_ref.shape[2]

    @pl.when(jnp.logical_and(b == 0, rt == 0))
    def _init_totals():
        acc_ref[3] = 0.0
        acc_ref[4] = 0.0

    @pl.when(rt == 0)
    def _init_group():
        acc_ref[0] = 0.0
        lab = lrow_ref[0]
        s = srow_ref[0]
        sumsq = jnp.zeros((), jnp.float32)
        lin = jnp.zeros((), jnp.float32)
        for a in range(_NUM_CLASSES):
            cnt = jnp.sum(jnp.where(lab == a, 1.0, 0.0))
            sumsq += cnt * cnt
            lin += cnt * jnp.sum(s * jnp.sign(lab - a).astype(jnp.float32))
        acc_ref[1] = (float(g * g) - sumsq) * 0.5
        acc_ref[2] = lin

    sca = sca_ref[...]
    scb = scb_ref[...]
    lca = lca_ref[...]
    lcb = lcb_ref[...]
    srow = srow_ref[0]
    srev = srev_ref[0]
    lrow = lrow_ref[0]
    lrev = lrev_ref[0]

    h = g // 2
    c1 = -1.4426950408889634
    c2 = 0.5 / 0.6931471805599453

    def t_of(a):
        return jnp.log2(1.0 + jnp.exp2(a * c1)) + c2 * a

    shape = (tr, h)
    r = rt * tr + jax.lax.broadcasted_iota(jnp.int32, shape, 0)
    j = jax.lax.broadcasted_iota(jnp.int32, shape, 1)
    top = j < r
    dl = jnp.where(top, sca - srow[:, :h], scb - srev[:, :h])
    ml = jnp.logical_or(jnp.logical_and(top, lca != lrow[:, :h]),
                        jnp.logical_and(jnp.logical_not(top),
                                        lcb != lrev[:, :h]))
    tot = jnp.sum(jnp.where(ml, t_of(jnp.abs(dl)), 0.0))
    dr = scb - srev[:, h:]
    mr = lcb != lrev[:, h:]
    tot += jnp.sum(jnp.where(mr, t_of(jnp.abs(dr)), 0.0))
    acc_ref[0] += tot

    @pl.when(rt == nt - 1)
    def _finalize_group():
        n_pairs = acc_ref[1]
        safe_n = jnp.where(n_pairs > 0, n_pairs, 1.0)
        ln2 = 0.6931471805599453
        loss = (ln2 * acc_ref[0] - 0.5 * acc_ref[2]) / safe_n
        acc_ref[3] += jnp.where(n_pairs > 0, loss, 0.0)
        acc_ref[4] += jnp.where(n_pairs > 0, 1.0, 0.0)

        @pl.when(b == nb - 1)
        def _finalize_output():
            count = acc_ref[4]
            safe_c = jnp.where(count > 0, count, 1.0)
            out_ref[0, 0] = jnp.where(count > 0, acc_ref[3] / safe_c, 0.0)


def kernel(scores, labels, group_sizes):
    scores = scores.reshape(-1)
    labels = labels.reshape(-1)
    n = scores.shape[0]
    num_groups = group_sizes.shape[0]
    g = n // num_groups
    h = g // 2
    tr = 512
    nt = h // tr

    s2 = scores.reshape(num_groups, g)
    l2 = labels.reshape(num_groups, g)
    sca = s2[:, :h].reshape(num_groups * h, 1)
    scb = s2[:, :h - 1:-1].reshape(num_groups * h, 1)
    lca = l2[:, :h].reshape(num_groups * h, 1)
    lcb = l2[:, :h - 1:-1].reshape(num_groups * h, 1)
    srow = s2.reshape(num_groups, 1, g)
    srev = s2[:, ::-1].reshape(num_groups, 1, g)
    lrow = l2.reshape(num_groups, 1, g)
    lrev = l2[:, ::-1].reshape(num_groups, 1, g)

    col = pl.BlockSpec((tr, 1), lambda b, rt: (b * nt + rt, 0))
    row = pl.BlockSpec((1, 1, g), lambda b, rt: (b, 0, 0))

    out = pl.pallas_call(
        _rank_loss_kernel,
        grid=(num_groups, nt),
        in_specs=[col, col, col, col, row, row, row, row],
        out_specs=pl.BlockSpec(memory_space=pltpu.SMEM),
        out_shape=jax.ShapeDtypeStruct((1, 1), jnp.float32),
        scratch_shapes=[pltpu.SMEM((5,), jnp.float32)],
    )(sca, scb, lca, lcb, srow, srev, lrow, lrev)
    return out[0, 0]
